# Initial kernel scaffold; baseline (speedup 1.0000x reference)
import jax
import jax.numpy as jnp
from jax.experimental import pallas as pl


def kernel(noisy_symbols, G, sigma2):
    raise NotImplementedError("write your pallas kernel here")


# speedup vs baseline: 8.9684x; 8.9684x over previous
import jax
import jax.numpy as jnp
from jax.experimental import pallas as pl

_N = 32
_K = 12
_W = 2 ** _K


def _decode_kernel(noisy_ref, g_ref, sig_ref, out_ref):
    x = noisy_ref[...] * (-4.0 / sig_ref[0, 0])

    gf = g_ref[...].astype(jnp.float32)
    w_ids = jax.lax.broadcasted_iota(jnp.int32, (_K, _W), 1)
    j_ids = jax.lax.broadcasted_iota(jnp.int32, (_K, _W), 0)
    bits_t = ((w_ids >> j_ids) & 1).astype(jnp.float32)
    c_t = jax.lax.dot_general(
        gf, bits_t, (((0,), (0,)), ((), ())),
        preferred_element_type=jnp.float32)
    c_t = c_t - 2.0 * jnp.floor(c_t * 0.5)
    s_t = 1.0 - 2.0 * c_t

    scores = jnp.dot(x, s_t, preferred_element_type=jnp.float32,
                     precision=jax.lax.Precision.HIGHEST)

    best = jnp.max(scores, axis=1, keepdims=True)
    wid = jax.lax.broadcasted_iota(jnp.int32, scores.shape, 1)
    idx = jnp.min(jnp.where(scores == best, wid, _W), axis=1,
                  keepdims=True)

    jbit = jax.lax.broadcasted_iota(jnp.int32, (scores.shape[0], _K), 1)
    out_ref[...] = ((idx >> jbit) & 1).astype(jnp.float32)


def kernel(noisy_symbols, G, sigma2):
    noisy = noisy_symbols.astype(jnp.float32)
    b = noisy.shape[0]
    sig = jnp.reshape(sigma2.astype(jnp.float32), (1, 1))
    return pl.pallas_call(
        _decode_kernel,
        out_shape=jax.ShapeDtypeStruct((b, _K), jnp.float32),
    )(noisy, G, sig)


# speedup vs baseline: 16.4180x; 1.8307x over previous
import jax
import jax.numpy as jnp
from jax.experimental import pallas as pl

_N = 32
_K = 12
_W = 2 ** _K


def _decode_kernel(noisy_ref, g_ref, sig_ref, out_ref):
    x = noisy_ref[...] * (-4.0 / sig_ref[0, 0])

    gf = g_ref[...].astype(jnp.float32)
    w_ids = jax.lax.broadcasted_iota(jnp.int32, (_K, _W), 1)
    j_ids = jax.lax.broadcasted_iota(jnp.int32, (_K, _W), 0)
    bits_t = ((w_ids >> j_ids) & 1).astype(jnp.float32)
    c_t = jax.lax.dot_general(
        gf, bits_t, (((0,), (0,)), ((), ())),
        preferred_element_type=jnp.float32)
    c_t = c_t - 2.0 * jnp.floor(c_t * 0.5)
    s_t = 1.0 - 2.0 * c_t

    x1 = x.astype(jnp.bfloat16)
    r1 = x - x1.astype(jnp.float32)
    x2 = r1.astype(jnp.bfloat16)
    x3 = (r1 - x2.astype(jnp.float32)).astype(jnp.bfloat16)
    xc = jnp.concatenate([x1, x2, x3], axis=1)
    s_bf = s_t.astype(jnp.bfloat16)
    sc = jnp.concatenate([s_bf, s_bf, s_bf], axis=0)
    scores = jnp.dot(xc, sc, preferred_element_type=jnp.float32)

    idx = jnp.argmax(scores, axis=1).astype(jnp.int32)[:, None]

    jbit = jax.lax.broadcasted_iota(jnp.int32, (scores.shape[0], _K), 1)
    out_ref[...] = ((idx >> jbit) & 1).astype(jnp.float32)


def kernel(noisy_symbols, G, sigma2):
    noisy = noisy_symbols.astype(jnp.float32)
    b = noisy.shape[0]
    sig = jnp.reshape(sigma2.astype(jnp.float32), (1, 1))
    return pl.pallas_call(
        _decode_kernel,
        out_shape=jax.ShapeDtypeStruct((b, _K), jnp.float32),
    )(noisy, G, sig)


# speedup vs baseline: 16.4887x; 1.0043x over previous
import jax
import jax.numpy as jnp
from jax.experimental import pallas as pl

_N = 32
_K = 12
_W = 2 ** _K


def _decode_kernel(noisy_ref, g_ref, sig_ref, out_ref):
    x = noisy_ref[...] * (-4.0 / sig_ref[0, 0])

    gf = g_ref[...].astype(jnp.float32)
    w_ids = jax.lax.broadcasted_iota(jnp.int32, (_K, _W), 1)
    j_ids = jax.lax.broadcasted_iota(jnp.int32, (_K, _W), 0)
    bits_t = ((w_ids >> j_ids) & 1).astype(jnp.float32)
    c_t = jax.lax.dot_general(
        gf, bits_t, (((0,), (0,)), ((), ())),
        preferred_element_type=jnp.float32)
    c_t = c_t - 2.0 * jnp.floor(c_t * 0.5)
    s_t = 1.0 - 2.0 * c_t

    x1 = x.astype(jnp.bfloat16)
    r1 = x - x1.astype(jnp.float32)
    x2 = r1.astype(jnp.bfloat16)
    x3 = (r1 - x2.astype(jnp.float32)).astype(jnp.bfloat16)
    xc = jnp.concatenate([x1, x2, x3], axis=1)
    s_bf = s_t.astype(jnp.bfloat16)
    sc = jnp.concatenate([s_bf, s_bf, s_bf], axis=0)
    scores = jnp.dot(xc, sc, preferred_element_type=jnp.float32)

    idx = jnp.argmax(scores, axis=1).astype(jnp.int32)[:, None]

    jbit = jax.lax.broadcasted_iota(jnp.int32, (scores.shape[0], _K), 1)
    out_ref[...] = ((idx >> jbit) & 1).astype(jnp.float32)


def kernel(noisy_symbols, G, sigma2):
    noisy = noisy_symbols
    b = noisy.shape[0]
    sig = jnp.reshape(sigma2.astype(jnp.float32), (1, 1))
    return pl.pallas_call(
        _decode_kernel,
        out_shape=jax.ShapeDtypeStruct((b, _K), jnp.float32),
    )(noisy, G, sig)
